# Initial kernel scaffold; baseline (speedup 1.0000x reference)
import jax
import jax.numpy as jnp
from jax.experimental import pallas as pl


def kernel(x, edge, edge_index, edge_attr, node_attr, additional_message_features, edge_dist_gauss, W_m1, Wa_m1, W_m2, Wa_m2, W_u1, Wa_u1, W_u2, Wa_u2, W_e1, Wa_e1, Wg1a, Wg2a, W_e2, Wa_e2, Wg1b, Wg2b):
    raise NotImplementedError("write your pallas kernel here")


# speedup vs baseline: 1.8472x; 1.8472x over previous
import functools

import jax
import jax.numpy as jnp
from jax import lax
from jax.experimental import pallas as pl
from jax.experimental.pallas import tpu as pltpu
from jax.experimental.pallas import tpu_sc as plsc

N = 10000
E = 160000
D = 256
DH = 128

NC = 2
NS = 16
NW = NC * NS

@functools.cache
def _mesh():
    return plsc.VectorSubcoreMesh(
        core_axis_name="c", subcore_axis_name="s", num_cores=NC, num_subcores=NS)


def _swish(v):
    return v * jax.nn.sigmoid(v)


_GC = 128
_PER_W = E // NW
_GN = _PER_W // _GC
_GT = _PER_W - _GN * _GC


def _gather_pair(yi, yj, dst, src):
    @functools.partial(
        pl.kernel,
        out_type=[jax.ShapeDtypeStruct((E, D), jnp.float32)] * 2,
        mesh=_mesh(),
        scratch_types=[
            pltpu.VMEM((_GC,), jnp.int32),
            pltpu.VMEM((_GC,), jnp.int32),
            pltpu.VMEM((_GC, D), jnp.float32),
            pltpu.VMEM((_GC, D), jnp.float32),
            pltpu.VMEM((_GT,), jnp.int32),
            pltpu.VMEM((_GT,), jnp.int32),
            pltpu.VMEM((_GT, D), jnp.float32),
            pltpu.VMEM((_GT, D), jnp.float32),
            pltpu.SemaphoreType.DMA,
        ],
    )
    def k(yi_h, yj_h, dst_h, src_h, oi_h, oj_h,
          id_v, is_v, bi_v, bj_v, id_t, is_t, bi_t, bj_t, sem):
        wid = lax.axis_index("s") * NC + lax.axis_index("c")
        w0 = pl.multiple_of(wid * _PER_W, 8)

        def chunk(base, idv, isv, bi, bj, sz):
            pltpu.sync_copy(dst_h.at[pl.ds(base, sz)], idv)
            pltpu.sync_copy(src_h.at[pl.ds(base, sz)], isv)
            pltpu.async_copy(yi_h.at[idv], bi, sem).wait()
            pltpu.async_copy(yj_h.at[isv], bj, sem).wait()
            pltpu.sync_copy(bi, oi_h.at[pl.ds(base, sz)])
            pltpu.sync_copy(bj, oj_h.at[pl.ds(base, sz)])

        @pl.loop(0, _GN)
        def _(t):
            chunk(pl.multiple_of(w0 + t * _GC, 8), id_v, is_v, bi_v, bj_v, _GC)

        chunk(pl.multiple_of(w0 + _GN * _GC, 8), id_t, is_t, bi_t, bj_t, _GT)

    return k(yi, yj, dst, src)


_SC_CH = 128
_PER_T = E // NS
_SN = _PER_T // _SC_CH
_ST = _PER_T - _SN * _SC_CH
_NP = 10240
_RPT = _NP // NS


def _segment_sum(m2s, dst, zeros_half):
    @functools.partial(
        pl.kernel,
        out_type=jax.ShapeDtypeStruct((NC, _NP, DH), jnp.float32),
        mesh=_mesh(),
        scratch_types=[
            pltpu.VMEM((_SC_CH,), jnp.int32),
            pltpu.VMEM((_SC_CH, DH), jnp.float32),
            pltpu.VMEM((_ST,), jnp.int32),
            pltpu.VMEM((_ST, DH), jnp.float32),
            pltpu.VMEM_SHARED((_NP, DH), jnp.float32),
            pltpu.SemaphoreType.DMA,
        ],
    )
    def k(m2s_h, dst_h, z_h, agg_h, idx_v, buf_v, idx_t, buf_t, acc_s, sem):
        c = lax.axis_index("c")
        tid = lax.axis_index("s")
        pltpu.sync_copy(z_h.at[pl.ds(tid * _RPT, _RPT)],
                        acc_s.at[pl.ds(tid * _RPT, _RPT)])
        plsc.subcore_barrier()

        e0 = pl.multiple_of(tid * _PER_T, 8)

        def chunk(base, idv, bufv, sz):
            pltpu.sync_copy(dst_h.at[pl.ds(base, sz)], idv)
            pltpu.sync_copy(m2s_h.at[c, pl.ds(base, sz), :], bufv)
            pltpu.sync_copy(bufv, acc_s.at[idv], add=True)

        @pl.loop(0, _SN)
        def _(t):
            chunk(pl.multiple_of(e0 + t * _SC_CH, 8), idx_v, buf_v, _SC_CH)

        chunk(pl.multiple_of(e0 + _SN * _SC_CH, 8), idx_t, buf_t, _ST)

        plsc.subcore_barrier()
        pltpu.sync_copy(acc_s.at[pl.ds(tid * _RPT, _RPT)],
                        agg_h.at[c, pl.ds(tid * _RPT, _RPT), :])

    return k(m2s, dst, zeros_half)


_BN = 2000
_BE = 640


def _dot(a, b):
    return jnp.dot(a, b, preferred_element_type=jnp.float32)


def _node_proj(x, wxi, wxj):
    def body(x_r, wi_r, wj_r, yi_r, yj_r):
        xb = x_r[...]
        yi_r[...] = _dot(xb, wi_r[...])
        yj_r[...] = _dot(xb, wj_r[...])

    full = lambda s: pl.BlockSpec(s, lambda i: (0, 0))
    return pl.pallas_call(
        body,
        grid=(N // _BN,),
        in_specs=[pl.BlockSpec((_BN, D), lambda i: (i, 0)), full((D, D)), full((D, D))],
        out_specs=[pl.BlockSpec((_BN, D), lambda i: (i, 0))] * 2,
        out_shape=[jax.ShapeDtypeStruct((N, D), jnp.float32)] * 2,
    )(x, wxi, wxj)


def _message(yi_g, yj_g, amf, edge, ea, w4, we, wa1, wm2, wa2):
    def body(yi_r, yj_r, amf_r, edge_r, ea_r, w4_r, we_r, wa1_r, wm2_r, wa2_r, out_r):
        eab = ea_r[...]
        t = yi_r[...] + yj_r[...] + _dot(amf_r[...], w4_r[...]) + _dot(edge_r[...], we_r[...])
        m1 = _swish(t * _dot(eab, wa1_r[...]))
        m2 = _swish(_dot(m1, wm2_r[...]) * _dot(eab, wa2_r[...]))
        out_r[0] = m2[:, :DH]
        out_r[1] = m2[:, DH:]

    full = lambda s: pl.BlockSpec(s, lambda i: tuple(0 for _ in s))
    return pl.pallas_call(
        body,
        grid=(E // _BE,),
        in_specs=[
            pl.BlockSpec((_BE, D), lambda i: (i, 0)),
            pl.BlockSpec((_BE, D), lambda i: (i, 0)),
            pl.BlockSpec((_BE, 4), lambda i: (i, 0)),
            pl.BlockSpec((_BE, D), lambda i: (i, 0)),
            pl.BlockSpec((_BE, 16), lambda i: (i, 0)),
            full((4, D)), full((D, D)), full((16, D)), full((D, D)), full((16, D)),
        ],
        out_specs=pl.BlockSpec((NC, _BE, DH), lambda i: (0, i, 0)),
        out_shape=jax.ShapeDtypeStruct((NC, E, DH), jnp.float32),
    )(yi_g, yj_g, amf, edge, ea, w4, we, wa1, wm2, wa2)


def _node_update(x, agg3, na, wu1a, wu1b, wau1, wu2, wau2, we1a, we1b):
    def body(x_r, ag_r, na_r, wu1a_r, wu1b_r, wau1_r, wu2_r, wau2_r,
             we1a_r, we1b_r, xn_r, ai_r, aj_r):
        xb = x_r[...]
        nab = na_r[...]
        agg = jnp.concatenate([ag_r[0], ag_r[1]], axis=-1)
        u = _swish((_dot(xb, wu1a_r[...]) + _dot(agg, wu1b_r[...]))
                   * _dot(nab, wau1_r[...]))
        u = _dot(u, wu2_r[...]) * _dot(nab, wau2_r[...])
        xn = xb + u
        xn_r[...] = xn
        ai_r[...] = _dot(xn, we1a_r[...])
        aj_r[...] = _dot(xn, we1b_r[...])

    full = lambda s: pl.BlockSpec(s, lambda i: tuple(0 for _ in s))
    return pl.pallas_call(
        body,
        grid=(N // _BN,),
        in_specs=[
            pl.BlockSpec((_BN, D), lambda i: (i, 0)),
            pl.BlockSpec((NC, _BN, DH), lambda i: (0, i, 0)),
            pl.BlockSpec((_BN, 16), lambda i: (i, 0)),
            full((D, D)), full((D, D)), full((16, D)),
            full((D, D)), full((16, D)), full((D, D)), full((D, D)),
        ],
        out_specs=[pl.BlockSpec((_BN, D), lambda i: (i, 0))] * 3,
        out_shape=[jax.ShapeDtypeStruct((N, D), jnp.float32)] * 3,
    )(x, agg3, na, wu1a, wu1b, wau1, wu2, wau2, we1a, we1b)


def _edge_update(ai_g, aj_g, edge, ea, g, wae1, wg1a, wg2a, we2, wae2, wg1b, wg2b):
    def body(ai_r, aj_r, edge_r, ea_r, g_r, wae1_r, wg1a_r, wg2a_r, we2_r, wae2_r,
             wg1b_r, wg2b_r, out_r):
        eab = ea_r[...]
        gb = g_r[...]
        wa = _dot(_swish(_dot(gb, wg1a_r[...])), wg2a_r[...])
        e1 = _swish((ai_r[...] + aj_r[...]) * _dot(eab, wae1_r[...]) * wa)
        wb = _dot(_swish(_dot(gb, wg1b_r[...])), wg2b_r[...])
        e2 = _swish(_dot(e1, we2_r[...]) * _dot(eab, wae2_r[...]) * wb)
        out_r[...] = edge_r[...] + e2

    full = lambda s: pl.BlockSpec(s, lambda i: tuple(0 for _ in s))
    return pl.pallas_call(
        body,
        grid=(E // _BE,),
        in_specs=[
            pl.BlockSpec((_BE, D), lambda i: (i, 0)),
            pl.BlockSpec((_BE, D), lambda i: (i, 0)),
            pl.BlockSpec((_BE, D), lambda i: (i, 0)),
            pl.BlockSpec((_BE, 16), lambda i: (i, 0)),
            pl.BlockSpec((_BE, 128), lambda i: (i, 0)),
            full((16, D)), full((128, 64)), full((64, D)), full((D, D)),
            full((16, D)), full((128, 64)), full((64, D)),
        ],
        out_specs=pl.BlockSpec((_BE, D), lambda i: (i, 0)),
        out_shape=jax.ShapeDtypeStruct((E, D), jnp.float32),
    )(ai_g, aj_g, edge, ea, g, wae1, wg1a, wg2a, we2, wae2, wg1b, wg2b)


def kernel(x, edge, edge_index, edge_attr, node_attr, additional_message_features,
           edge_dist_gauss, W_m1, Wa_m1, W_m2, Wa_m2, W_u1, Wa_u1, W_u2, Wa_u2,
           W_e1, Wa_e1, Wg1a, Wg2a, W_e2, Wa_e2, Wg1b, Wg2b):
    src = edge_index[0]
    dst = edge_index[1]
    zeros_half = jnp.zeros((_NP, DH), dtype=jnp.float32)

    w4 = W_m1[:4]
    yi, yj = _node_proj(x, W_m1[4:4 + D], W_m1[4 + D:4 + 2 * D])
    yi_g, yj_g = _gather_pair(yi, yj, dst, src)
    m2s = _message(yi_g, yj_g, additional_message_features, edge, edge_attr,
                   w4, W_m1[4 + 2 * D:], Wa_m1, W_m2, Wa_m2)
    agg3 = _segment_sum(m2s, dst, zeros_half)[:, :N, :]

    x_new, ai, aj = _node_update(x, agg3, node_attr, W_u1[:D], W_u1[D:],
                                 Wa_u1, W_u2, Wa_u2, W_e1[:D], W_e1[D:])

    ai_g, aj_g = _gather_pair(ai, aj, dst, src)
    edge_new = _edge_update(ai_g, aj_g, edge, edge_attr, edge_dist_gauss,
                            Wa_e1, Wg1a, Wg2a, W_e2, Wa_e2, Wg1b, Wg2b)
    return (x_new, edge_new)
